# Initial kernel scaffold; baseline (speedup 1.0000x reference)
import jax
import jax.numpy as jnp
from jax.experimental import pallas as pl


def kernel(params, input_ids, attention_mask):
    raise NotImplementedError("write your pallas kernel here")


# speedup vs baseline: 1.2344x; 1.2344x over previous
import functools

import jax
import jax.numpy as jnp
from jax import lax
from jax.experimental import pallas as pl
from jax.experimental.pallas import tpu as pltpu
from jax.experimental.pallas import tpu_sc as plsc

V = 50257
D = 768
L = 2
H = 12
HD = D // H
FF = 3072
E = 8
K = 2
B = 1
S = 2048

BQ = 512
BLK = 128
P_PAD = S * K + E * BLK
NBLK = P_PAD // BLK
TB = 256
VB = 1024


HI = lax.Precision.HIGHEST


def _mm(a, b):
    return jnp.matmul(a, b, precision=HI)


def _mmb(a, b):
    return jnp.matmul(a.astype(jnp.bfloat16), b.astype(jnp.bfloat16),
                      preferred_element_type=jnp.float32)


def _mmb_t(a, b):
    return lax.dot_general(a.astype(jnp.bfloat16), b.astype(jnp.bfloat16),
                           (((0,), (0,)), ((), ())),
                           preferred_element_type=jnp.float32)


def _ln(x, g, b):
    m = jnp.mean(x, axis=-1, keepdims=True)
    v = jnp.mean((x - m) ** 2, axis=-1, keepdims=True)
    return (x - m) / jnp.sqrt(v + 1e-5) * g + b


def _gelu(x):
    return 0.5 * x * (1.0 + lax.erf(x * 0.7071067811865476))


def _embed_gather(table, ids):
    info = plsc.get_sparse_core_info()
    nw = info.num_cores * info.num_subcores
    bpw = S // nw
    mesh = plsc.VectorSubcoreMesh(core_axis_name="c", subcore_axis_name="s")

    @functools.partial(
        pl.kernel,
        mesh=mesh,
        out_type=jax.ShapeDtypeStruct((S, D), jnp.float32),
        scratch_types=[
            pltpu.VMEM((bpw,), jnp.int32),
            pltpu.VMEM((bpw, D), jnp.float32),
            pltpu.SemaphoreType.DMA,
        ],
    )
    def k(table_hbm, idx_hbm, out_hbm, idx_v, rows_v, sem):
        wid = lax.axis_index("s") * info.num_cores + lax.axis_index("c")
        base = wid * bpw
        pltpu.sync_copy(idx_hbm.at[pl.ds(base, bpw)], idx_v)
        pltpu.async_copy(table_hbm.at[idx_v], rows_v, sem).wait()
        pltpu.sync_copy(rows_v, out_hbm.at[pl.ds(base, bpw)])

    return k(table, ids)


def _qkv(xs, g, b, wq, bq, wk, bk, wv, bv):
    n = len(xs)

    def body(*refs):
        x = refs[0][...]
        for r in refs[1:n]:
            x = x + r[...]
        (g_ref, b_ref, wq_ref, bq_ref, wk_ref, bk_ref, wv_ref, bv_ref,
         q_ref, k_ref, v_ref) = refs[n:]
        h = _ln(x, g_ref[...], b_ref[...])
        q_ref[...] = _mmb(h, wq_ref[...]) + bq_ref[...]
        k_ref[...] = _mmb(h, wk_ref[...]) + bk_ref[...]
        v_ref[...] = _mmb(h, wv_ref[...]) + bv_ref[...]

    sb = 512
    out = [jax.ShapeDtypeStruct((S, D), jnp.float32)] * 3
    row = pl.BlockSpec((sb, D), lambda i: (i, 0))
    full = pl.BlockSpec((D, D), lambda i: (0, 0))
    vec = pl.BlockSpec((1, D), lambda i: (0, 0))
    return pl.pallas_call(
        body,
        grid=(S // sb,),
        in_specs=[row] * n + [vec, vec, full, vec, full, vec, full, vec],
        out_specs=[row, row, row],
        out_shape=out)(*xs, g, b, wq, bq, wk, bk, wv, bv)


def _attn_body(q_ref, k_ref, v_ref, cq_ref, sq_ref, ck_ref, sk_ref, o_ref):
    qi = pl.program_id(1)

    def rope(x, c, s):
        x1 = x[:, :HD // 2]
        x2 = x[:, HD // 2:]
        return x * c + jnp.concatenate([-x2, x1], axis=1) * s

    qr = rope(q_ref[0], cq_ref[...], sq_ref[...])
    kr = rope(k_ref[0], ck_ref[...], sk_ref[...])
    s = lax.dot_general(qr.astype(jnp.bfloat16), kr.astype(jnp.bfloat16),
                        (((1,), (1,)), ((), ())),
                        preferred_element_type=jnp.float32) * (1.0 / 8.0)
    row = lax.broadcasted_iota(jnp.int32, (BQ, S), 0) + qi * BQ
    col = lax.broadcasted_iota(jnp.int32, (BQ, S), 1)
    s = jnp.where(col > row, -1e9, s)
    m = jnp.max(s, axis=-1, keepdims=True)
    p = jnp.exp(s - m)
    p = p * (1.0 / jnp.sum(p, axis=-1, keepdims=True))
    o_ref[0] = _mmb(p, v_ref[0])


def _attn(q, k, v, cos, sin):
    grid = (H, S // BQ)
    return pl.pallas_call(
        _attn_body,
        grid=grid,
        in_specs=[
            pl.BlockSpec((1, BQ, HD), lambda h, qi: (h, qi, 0)),
            pl.BlockSpec((1, S, HD), lambda h, qi: (h, 0, 0)),
            pl.BlockSpec((1, S, HD), lambda h, qi: (h, 0, 0)),
            pl.BlockSpec((BQ, HD), lambda h, qi: (qi, 0)),
            pl.BlockSpec((BQ, HD), lambda h, qi: (qi, 0)),
            pl.BlockSpec((S, HD), lambda h, qi: (0, 0)),
            pl.BlockSpec((S, HD), lambda h, qi: (0, 0)),
        ],
        out_specs=pl.BlockSpec((1, BQ, HD), lambda h, qi: (h, qi, 0)),
        out_shape=jax.ShapeDtypeStruct((H, S, HD), jnp.float32),
    )(q, k, v, cos, sin, cos, sin)


def _proj(xs, o, wo, bo):
    n = len(xs)

    def body(*refs):
        x = refs[0][...]
        for r in refs[1:n]:
            x = x + r[...]
        o_ref, wo_ref, bo_ref, x2_ref = refs[n:]
        x2_ref[...] = x + _mmb(o_ref[...], wo_ref[...]) + bo_ref[...]

    sb = 512
    row = pl.BlockSpec((sb, D), lambda i: (i, 0))
    return pl.pallas_call(
        body,
        grid=(S // sb,),
        in_specs=[row] * n + [row, pl.BlockSpec((D, D), lambda i: (0, 0)),
                              pl.BlockSpec((1, D), lambda i: (0, 0))],
        out_specs=row,
        out_shape=jax.ShapeDtypeStruct((S, D), jnp.float32))(*xs, o, wo, bo)


def _post_body(x2_ref, g2_ref, b2_ref, rw_ref, rb_ref,
               h_ref, p0_ref, p1_ref, g0_ref, g1_ref, p0i_ref, p1i_ref,
               be_ref, nb_ref):
    h = _ln(x2_ref[...], g2_ref[...], b2_ref[...])
    h_ref[...] = h

    rl = _mmb(h, rw_ref[...]) + rb_ref[...]
    m = jnp.max(rl, axis=-1, keepdims=True)
    ex = jnp.exp(rl - m)
    pr = ex / jnp.sum(ex, axis=-1, keepdims=True)

    lane = lax.broadcasted_iota(jnp.int32, (S, E), 1)
    m0 = jnp.max(pr, axis=-1, keepdims=True)
    i0 = jnp.min(jnp.where(pr == m0, lane, E), axis=-1, keepdims=True)
    pr1 = jnp.where(lane == i0, -1.0, pr)
    m1 = jnp.max(pr1, axis=-1, keepdims=True)
    i1 = jnp.min(jnp.where(pr1 == m1, lane, E), axis=-1, keepdims=True)
    ssum = m0 + m1 + 1e-9
    g0_ref[...] = m0 / ssum
    g1_ref[...] = m1 / ssum

    oh0 = (lane == i0).astype(jnp.float32)
    oh1 = (lane == i1).astype(jnp.float32)

    ch = 512
    ri = lax.broadcasted_iota(jnp.int32, (ch, ch), 0)
    ci = lax.broadcasted_iota(jnp.int32, (ch, ch), 1)
    tri = (ci < ri).astype(jnp.float32)
    carry0 = jnp.zeros((1, E), jnp.float32)
    carry1 = jnp.zeros((1, E), jnp.float32)
    r0s, r1s = [], []
    for c in range(S // ch):
        o0 = oh0[c * ch:(c + 1) * ch]
        o1 = oh1[c * ch:(c + 1) * ch]
        r0s.append(_mmb(tri, o0) + carry0)
        r1s.append(_mmb(tri, o1) + carry1)
        carry0 = carry0 + jnp.sum(o0, axis=0, keepdims=True)
        carry1 = carry1 + jnp.sum(o1, axis=0, keepdims=True)
    r0 = jnp.concatenate(r0s, axis=0)
    r1 = jnp.concatenate(r1s, axis=0) + carry0
    cnt = carry0 + carry1

    pc = jnp.ceil(cnt * (1.0 / BLK)) * BLK
    er = lax.broadcasted_iota(jnp.int32, (E, E), 0)
    ec = lax.broadcasted_iota(jnp.int32, (E, E), 1)
    slt = (er < ec).astype(jnp.float32)
    offs = _mmb(pc, slt)

    p0v = jnp.sum(oh0 * (r0 + offs), axis=-1, keepdims=True)
    p1v = jnp.sum(oh1 * (r1 + offs), axis=-1, keepdims=True)
    p0_ref[...] = p0v
    p1_ref[...] = p1v
    p0i_ref[...] = p0v.astype(jnp.int32)
    p1i_ref[...] = p1v.astype(jnp.int32)

    ends = offs + pc
    bs = lax.broadcasted_iota(jnp.int32, (NBLK, E), 0).astype(jnp.float32) * BLK
    be = jnp.sum((jnp.broadcast_to(ends, (NBLK, E)) <= bs).astype(jnp.int32),
                 axis=-1, keepdims=True)
    be_ref[...] = jnp.minimum(be, E - 1)
    total = jnp.sum(pc, axis=-1, keepdims=True)
    nb_ref[...] = (total * (1.0 / BLK)).astype(jnp.int32)


def _post(x2, g2, b2, rw, rb):
    out = [
        jax.ShapeDtypeStruct((S, D), jnp.float32),
        jax.ShapeDtypeStruct((S, 1), jnp.float32),
        jax.ShapeDtypeStruct((S, 1), jnp.float32),
        jax.ShapeDtypeStruct((S, 1), jnp.float32),
        jax.ShapeDtypeStruct((S, 1), jnp.float32),
        jax.ShapeDtypeStruct((S, 1), jnp.int32),
        jax.ShapeDtypeStruct((S, 1), jnp.int32),
        jax.ShapeDtypeStruct((NBLK, 1), jnp.int32),
        jax.ShapeDtypeStruct((1, 1), jnp.int32),
    ]
    return pl.pallas_call(_post_body, out_shape=out)(x2, g2, b2, rw, rb)


def _ffn_body(be_ref, nb_ref, h_ref, p0_ref, p1_ref, g0_ref, g1_ref,
              w1_ref, b1_ref, w2_ref, b2_ref, eo_ref):
    i = pl.program_id(0)

    @pl.when(i < nb_ref[0])
    def _():
        pos = (lax.broadcasted_iota(jnp.int32, (S, BLK), 1)
               + i * BLK).astype(jnp.float32)
        m0 = (p0_ref[...] == pos).astype(jnp.float32)
        m1 = (p1_ref[...] == pos).astype(jnp.float32)
        oh = m0 + m1
        ohg = m0 * g0_ref[...] + m1 * g1_ref[...]
        rows = _mmb_t(oh, h_ref[...])
        gates = _mmb_t(ohg, jnp.ones((S, 1), jnp.float32))
        mid = _gelu(_mmb(rows, w1_ref[0]) + b1_ref[0])
        out = _mmb(mid, w2_ref[0]) + b2_ref[0]
        out = out.astype(jnp.bfloat16).astype(jnp.float32)
        eo_ref[...] = out * gates

    @pl.when(i >= nb_ref[0])
    def _():
        eo_ref[...] = jnp.zeros((BLK, D), jnp.float32)


def _ffn(h, p0, p1, g0, g1, w1, b1, w2, b2, be, nb):
    grid_spec = pltpu.PrefetchScalarGridSpec(
        num_scalar_prefetch=2,
        grid=(NBLK,),
        in_specs=[
            pl.BlockSpec((S, D), lambda i, be, nb: (0, 0)),
            pl.BlockSpec((S, 1), lambda i, be, nb: (0, 0)),
            pl.BlockSpec((S, 1), lambda i, be, nb: (0, 0)),
            pl.BlockSpec((S, 1), lambda i, be, nb: (0, 0)),
            pl.BlockSpec((S, 1), lambda i, be, nb: (0, 0)),
            pl.BlockSpec((1, D, FF), lambda i, be, nb: (be[i], 0, 0)),
            pl.BlockSpec((1, 1, FF), lambda i, be, nb: (be[i], 0, 0)),
            pl.BlockSpec((1, FF, D), lambda i, be, nb: (be[i], 0, 0)),
            pl.BlockSpec((1, 1, D), lambda i, be, nb: (be[i], 0, 0)),
        ],
        out_specs=pl.BlockSpec((BLK, D), lambda i, be, nb: (i, 0)),
    )
    return pl.pallas_call(
        _ffn_body,
        grid_spec=grid_spec,
        out_shape=jax.ShapeDtypeStruct((P_PAD, D), jnp.float32),
    )(be, nb, h, p0, p1, g0, g1, w1, b1, w2, b2)


def _moe_gather(eo, p0i, p1i):
    info = plsc.get_sparse_core_info()
    nw = info.num_cores * info.num_subcores
    bpw = S // nw
    mesh = plsc.VectorSubcoreMesh(core_axis_name="c", subcore_axis_name="s")

    @functools.partial(
        pl.kernel,
        mesh=mesh,
        out_type=[jax.ShapeDtypeStruct((S, D), jnp.float32)] * 2,
        scratch_types=[
            pltpu.VMEM((bpw,), jnp.int32),
            pltpu.VMEM((bpw,), jnp.int32),
            pltpu.VMEM((bpw, D), jnp.float32),
            pltpu.VMEM((bpw, D), jnp.float32),
            pltpu.SemaphoreType.DMA,
            pltpu.SemaphoreType.DMA,
        ],
    )
    def k(eo_hbm, p0_hbm, p1_hbm, r0_hbm, r1_hbm,
          i0v, i1v, rows0, rows1, s0, s1):
        wid = lax.axis_index("s") * info.num_cores + lax.axis_index("c")
        base = wid * bpw
        pltpu.sync_copy(p0_hbm.at[pl.ds(base, bpw)], i0v)
        pltpu.sync_copy(p1_hbm.at[pl.ds(base, bpw)], i1v)
        c0 = pltpu.async_copy(eo_hbm.at[i0v], rows0, s0)
        c1 = pltpu.async_copy(eo_hbm.at[i1v], rows1, s1)
        c0.wait()
        c1.wait()
        pltpu.sync_copy(rows0, r0_hbm.at[pl.ds(base, bpw)])
        pltpu.sync_copy(rows1, r1_hbm.at[pl.ds(base, bpw)])

    return k(eo, p0i, p1i)


def _head(xs, g, b, w):
    n = len(xs)

    def body(*refs):
        x = refs[0][...]
        for r in refs[1:n]:
            x = x + r[...]
        g_ref, b_ref, w_ref, out_ref = refs[n:]
        xf = _ln(x, g_ref[...], b_ref[...])
        out_ref[...] = _mmb(xf, w_ref[...])

    nv = pl.cdiv(V, VB)
    sb = 1024
    row = pl.BlockSpec((sb, D), lambda i, j: (i, 0))
    return pl.pallas_call(
        body,
        grid=(S // sb, nv),
        in_specs=[row] * n + [
            pl.BlockSpec((1, D), lambda i, j: (0, 0)),
            pl.BlockSpec((1, D), lambda i, j: (0, 0)),
            pl.BlockSpec((D, VB), lambda i, j: (0, j)),
        ],
        out_specs=pl.BlockSpec((sb, VB), lambda i, j: (i, j)),
        out_shape=jax.ShapeDtypeStruct((S, V), jnp.float32),
    )(*xs, g, b, w)


def kernel(params, input_ids, attention_mask):
    p = params
    ids = input_ids.reshape(S).astype(jnp.int32)
    x = p['embed'][ids]

    t = jnp.arange(S, dtype=jnp.float32)
    inv = 1.0 / (10000.0 ** (jnp.arange(0, HD, 2, dtype=jnp.float32) / HD))
    fr = t[:, None] * inv[None, :]
    emb = jnp.concatenate((fr, fr), axis=-1)
    cos = jnp.cos(emb)
    sin = jnp.sin(emb)

    xs = (x,)
    for l in range(L):
        q, k, v = _qkv(xs, p['ln1_g'][l][None], p['ln1_b'][l][None],
                       p['wq'][l], p['bq'][l][None], p['wk'][l], p['bk'][l][None],
                       p['wv'][l], p['bv'][l][None])
        q = q.reshape(S, H, HD).transpose(1, 0, 2)
        k = k.reshape(S, H, HD).transpose(1, 0, 2)
        v = v.reshape(S, H, HD).transpose(1, 0, 2)
        o = _attn(q, k, v, cos, sin)
        o = o.transpose(1, 0, 2).reshape(S, D)
        x2 = _proj(xs, o, p['wo'][l], p['bo'][l][None])
        h, p0, p1, g0, g1, p0i, p1i, be, nb = _post(
            x2, p['ln2_g'][l][None], p['ln2_b'][l][None],
            p['rw'][l], p['rb'][l][None])
        eo = _ffn(h, p0, p1, g0, g1,
                  p['ew1'][l], p['eb1'][l].reshape(E, 1, FF),
                  p['ew2'][l], p['eb2'][l].reshape(E, 1, D),
                  be.reshape(NBLK), nb.reshape(1))
        r0, r1 = _moe_gather(eo, p0i.reshape(S), p1i.reshape(S))
        xs = (x2, r0, r1)

    logits = _head(xs, p['lnf_g'][None], p['lnf_b'][None], p['lm_head'])
    return logits.reshape(B, S, V)
